# baseline (speedup 1.0000x reference)
import functools

import jax
import jax.numpy as jnp
from jax import lax
from jax.experimental import pallas as pl
from jax.experimental.pallas import tpu as pltpu
from jax.experimental.pallas import tpu_sc as plsc

_B, _S = 4096, 200
_N = _B * _S
_NC, _NS = 2, 16
_NW = _NC * _NS
_PER_W = _N // _NW
_C = 128
_CHUNKS = _PER_W // _C
_NBLK = _N // 128


def _pad_relayout(pt):
    def body(x_ref, o_ref):
        t = jnp.transpose(x_ref[...].reshape(64, 4000))
        o_ref[...] = jnp.concatenate(
            [t, jnp.zeros((4000, 64), jnp.float32)], axis=1)

    return pl.pallas_call(
        body,
        grid=(250,),
        in_specs=[pl.BlockSpec((64, 1, 8, 500), lambda i: (0, i, 0, 0))],
        out_specs=pl.BlockSpec((4000, 128), lambda i: (i, 0)),
        out_shape=jax.ShapeDtypeStruct((1000000, 128), jnp.float32),
    )(pt.reshape(64, 250, 8, 500))


def _cont_proj(xv, wa, wb, b4):
    def body(x_ref, wa_ref, wb_ref, b_ref, o_ref):
        x = x_ref[...].reshape(3200, 128)
        xt = jnp.transpose(x)
        x8 = xt.reshape(3200, 128)
        e = jnp.dot(x8, wa_ref[...], preferred_element_type=jnp.float32)
        o = jnp.dot(x8, wb_ref[...], preferred_element_type=jnp.float32)
        inter = jnp.concatenate([e[:, None, :], o[:, None, :]], axis=1)
        o_ref[...] = inter.reshape(6400, 128) + b_ref[...]

    return pl.pallas_call(
        body,
        grid=(_B // 128,),
        in_specs=[
            pl.BlockSpec((200, 16, 128), lambda i: (0, 0, i)),
            pl.BlockSpec((128, 128), lambda i: (0, 0)),
            pl.BlockSpec((128, 128), lambda i: (0, 0)),
            pl.BlockSpec((1, 128), lambda i: (0, 0)),
        ],
        out_specs=pl.BlockSpec((6400, 128), lambda i: (i, 0)),
        out_shape=jax.ShapeDtypeStruct((_N // 4, 128), jnp.float32),
    )(xv, wa, wb, b4)


def _sc_assemble(pids, cids, ce4, p0t, p1t, c0t, c1t):
    mesh = plsc.VectorSubcoreMesh(core_axis_name="c", subcore_axis_name="s")

    @functools.partial(
        pl.kernel,
        mesh=mesh,
        compiler_params=pltpu.CompilerParams(use_tc_tiling_on_sc=False),
        out_type=jax.ShapeDtypeStruct((_N, 128), jnp.float32),
        scratch_types=[
            pltpu.VMEM((200, 64), jnp.float32),
            pltpu.VMEM((200, 64), jnp.float32),
            [pltpu.VMEM((2, 1, 128), jnp.int32)] * 8,
            [pltpu.VMEM((2, 1, 128), jnp.int32)] * 8,
            [pltpu.VMEM((_C, 128), jnp.float32)] * 4,
            [pltpu.VMEM((_C, 32), jnp.float32)] * 4,
            [pltpu.VMEM((_C, 32), jnp.float32)] * 4,
            [pltpu.SemaphoreType.DMA] * 8,
            [pltpu.SemaphoreType.DMA] * 4,
            [pltpu.SemaphoreType.DMA] * 4,
        ],
    )
    def k(pids_h, cids_h, ce_h, p0t_h, p1t_h, c0t_h, c1t_h,
          out_h, p0v, p1v, pb, cb, c0b, c1b, ceb, semi, semb, semc):
        wid = lax.axis_index("s") * _NC + lax.axis_index("c")
        base = wid * _PER_W
        bbase = wid * (_PER_W // 128)
        pltpu.sync_copy(p0t_h, p0v)
        pltpu.sync_copy(p1t_h, p1v)

        def issue_a(c, s8):
            blk = bbase + c
            pltpu.async_copy(pids_h.at[:, pl.ds(blk, 1), :], pb[s8], semi[s8])
            pltpu.async_copy(cids_h.at[:, pl.ds(blk, 1), :], cb[s8], semi[s8])

        def wait_a(s8):
            pltpu.make_async_copy(
                pids_h.at[:, pl.ds(0, 1), :], pb[s8], semi[s8]).wait()
            pltpu.make_async_copy(
                cids_h.at[:, pl.ds(0, 1), :], cb[s8], semi[s8]).wait()

        def wait_c(s4):
            pltpu.make_async_copy(
                c0b[s4].at[:, pl.ds(0, 64)],
                out_h.at[pl.ds(0, _C), pl.ds(0, 64)], semc[s4]).wait()
            pltpu.make_async_copy(
                c1b[s4], out_h.at[pl.ds(0, _C), pl.ds(64, 32)],
                semc[s4]).wait()
            pltpu.make_async_copy(
                ceb[s4], out_h.at[pl.ds(0, _C), pl.ds(96, 32)],
                semc[s4]).wait()

        def issue_b(c, s8, s4):
            @pl.when(c >= 4)
            def _():
                wait_c(s4)
            wait_a(s8)
            pltpu.async_copy(c0t_h.at[cb[s8].at[0, 0]], c0b[s4], semb[s4])
            pltpu.async_copy(c1t_h.at[cb[s8].at[1, 0]], c1b[s4], semb[s4])
            pltpu.async_copy(
                ce_h.at[pl.ds(base + c * _C, _C)], ceb[s4], semb[s4])

        def wait_b(s4):
            pltpu.make_async_copy(
                c0t_h.at[cb[0].at[0, 0]], c0b[s4], semb[s4]).wait()
            pltpu.make_async_copy(
                c1t_h.at[cb[0].at[1, 0]], c1b[s4], semb[s4]).wait()
            pltpu.make_async_copy(
                ce_h.at[pl.ds(0, _C)], ceb[s4], semb[s4]).wait()

        def compute(c, s8, s4):
            wait_b(s4)

            def grp(g2, c2):
                l0 = pl.multiple_of(g2 * 16, 16)
                i0vec = pb[s8][0, 0, pl.ds(l0, 16)]
                i1vec = pb[s8][1, 0, pl.ds(l0, 16)]
                for l in range(16):
                    t = l0 + l
                    id0 = i0vec[l]
                    id1 = i1vec[l]
                    for v in range(4):
                        plsc.addupdate(
                            c0b[s4].at[t, pl.ds(16 * v, 16)],
                            p0v[id0, pl.ds(16 * v, 16)])
                    for v in range(2):
                        plsc.addupdate(
                            c1b[s4].at[t, pl.ds(16 * v, 16)],
                            p1v[id1, pl.ds(16 * v, 16)])
                    for v in range(2):
                        plsc.addupdate(
                            ceb[s4].at[t, pl.ds(16 * v, 16)],
                            p1v[id1, pl.ds(32 + 16 * v, 16)])
                return c2

            lax.fori_loop(0, 8, grp, 0)
            off = base + c * _C
            pltpu.async_copy(
                c0b[s4].at[:, pl.ds(0, 64)],
                out_h.at[pl.ds(off, _C), pl.ds(0, 64)], semc[s4])
            pltpu.async_copy(
                c1b[s4], out_h.at[pl.ds(off, _C), pl.ds(64, 32)], semc[s4])
            pltpu.async_copy(
                ceb[s4], out_h.at[pl.ds(off, _C), pl.ds(96, 32)], semc[s4])

        for c0 in range(4):
            issue_a(c0, c0)
        issue_b(0, 0, 0)
        issue_b(1, 1, 1)

        def step(i, carry):
            for kk in range(8):
                c = 8 * i + kk

                @pl.when(c + 4 < _CHUNKS)
                def _():
                    issue_a(c + 4, (kk + 4) % 8)

                @pl.when(c + 2 < _CHUNKS)
                def _():
                    issue_b(c + 2, (kk + 2) % 8, (kk + 2) % 4)

                compute(c, kk, kk % 4)
            return carry

        lax.fori_loop(0, _CHUNKS // 8, step, 0)
        for s4 in range(4):
            wait_c(s4)

    return k(pids, cids, ce4, p0t, p1t, c0t, c1t)


def kernel(pos_ids_0, pos_ids_1, cat_ids_0, cat_ids_1, continuous_feature,
           pos_table_0, pos_table_1, cat_table_0, cat_table_1, W_cont, b_cont):
    pids = jnp.stack([pos_ids_0.reshape(_N), pos_ids_1.reshape(_N)]
                     ).astype(jnp.int32).reshape(2, _NBLK, 128)
    cids = jnp.stack([cat_ids_0.reshape(_N), cat_ids_1.reshape(_N)]
                     ).astype(jnp.int32).reshape(2, _NBLK, 128)
    ea = jnp.eye(8, 4, dtype=jnp.float32)
    eb = jnp.eye(8, 4, k=-4, dtype=jnp.float32)
    w = W_cont.astype(jnp.float32)
    wa = jnp.einsum("sv,kj->skvj", ea, w).reshape(128, 128)
    wb = jnp.einsum("sv,kj->skvj", eb, w).reshape(128, 128)
    b4 = jnp.tile(b_cont.astype(jnp.float32), 4).reshape(1, 128)
    ce4 = _cont_proj(continuous_feature.transpose(1, 2, 0), wa, wb, b4)
    c0pad = _pad_relayout(cat_table_0.T)
    out = _sc_assemble(pids, cids, ce4.reshape(_N, 32),
                       pos_table_0, pos_table_1, c0pad, cat_table_1)
    return out.reshape(_B, _S, 128)


# speedup vs baseline: 1.3895x; 1.3895x over previous
import functools

import jax
import jax.numpy as jnp
from jax import lax
from jax.experimental import pallas as pl
from jax.experimental.pallas import tpu as pltpu
from jax.experimental.pallas import tpu_sc as plsc

_B, _S = 4096, 200
_N = _B * _S
_NC, _NS = 2, 16
_NW = _NC * _NS
_PER_W = _N // _NW
_C = 128
_CHUNKS = _PER_W // _C
_NBLK = _N // 128


def _cont_proj(xv, wa, wb, b4):
    def body(x_ref, wa_ref, wb_ref, b_ref, o_ref):
        x = x_ref[...].reshape(3200, 128)
        xt = jnp.transpose(x)
        x8 = xt.reshape(3200, 128)
        e = jnp.dot(x8, wa_ref[...], preferred_element_type=jnp.float32)
        o = jnp.dot(x8, wb_ref[...], preferred_element_type=jnp.float32)
        inter = jnp.concatenate([e[:, None, :], o[:, None, :]], axis=1)
        o_ref[...] = inter.reshape(6400, 128) + b_ref[...]

    return pl.pallas_call(
        body,
        grid=(_B // 128,),
        in_specs=[
            pl.BlockSpec((200, 16, 128), lambda i: (0, 0, i)),
            pl.BlockSpec((128, 128), lambda i: (0, 0)),
            pl.BlockSpec((128, 128), lambda i: (0, 0)),
            pl.BlockSpec((1, 128), lambda i: (0, 0)),
        ],
        out_specs=pl.BlockSpec((6400, 128), lambda i: (i, 0)),
        out_shape=jax.ShapeDtypeStruct((_N // 4, 128), jnp.float32),
    )(xv, wa, wb, b4)


def _sc_assemble(pids, cids, ce4, p0t, p1t, c0t, c1t):
    mesh = plsc.VectorSubcoreMesh(core_axis_name="c", subcore_axis_name="s")

    @functools.partial(
        pl.kernel,
        mesh=mesh,
        compiler_params=pltpu.CompilerParams(use_tc_tiling_on_sc=False),
        out_type=jax.ShapeDtypeStruct((_N, 128), jnp.float32),
        scratch_types=[
            pltpu.VMEM((200, 64), jnp.float32),
            pltpu.VMEM((200, 64), jnp.float32),
            [pltpu.VMEM((2, 1, 128), jnp.int32)] * 8,
            [pltpu.VMEM((2, 1, 128), jnp.int32)] * 8,
            [pltpu.VMEM((_C, 64), jnp.float32)] * 4,
            [pltpu.VMEM((_C, 32), jnp.float32)] * 4,
            [pltpu.VMEM((_C, 32), jnp.float32)] * 4,
            [pltpu.SemaphoreType.DMA] * 8,
            [pltpu.SemaphoreType.DMA] * 4,
            [pltpu.SemaphoreType.DMA] * 4,
        ],
    )
    def k(pids_h, cids_h, ce_h, p0t_h, p1t_h, c0t_h, c1t_h,
          out_h, p0v, p1v, pb, cb, c0b, c1b, ceb, semi, semb, semc):
        wid = lax.axis_index("s") * _NC + lax.axis_index("c")
        base = wid * _PER_W
        bbase = wid * (_PER_W // 128)
        pltpu.sync_copy(p0t_h, p0v)
        pltpu.sync_copy(p1t_h, p1v)

        def issue_a(c, s8):
            blk = bbase + c
            pltpu.async_copy(pids_h.at[:, pl.ds(blk, 1), :], pb[s8], semi[s8])
            pltpu.async_copy(cids_h.at[:, pl.ds(blk, 1), :], cb[s8], semi[s8])

        def wait_a(s8):
            pltpu.make_async_copy(
                pids_h.at[:, pl.ds(0, 1), :], pb[s8], semi[s8]).wait()
            pltpu.make_async_copy(
                cids_h.at[:, pl.ds(0, 1), :], cb[s8], semi[s8]).wait()

        def wait_c(s4):
            pltpu.make_async_copy(
                c0b[s4], out_h.at[pl.ds(0, _C), pl.ds(0, 64)], semc[s4]).wait()
            pltpu.make_async_copy(
                c1b[s4], out_h.at[pl.ds(0, _C), pl.ds(64, 32)],
                semc[s4]).wait()
            pltpu.make_async_copy(
                ceb[s4], out_h.at[pl.ds(0, _C), pl.ds(96, 32)],
                semc[s4]).wait()

        def issue_b(c, s8, s4):
            @pl.when(c >= 4)
            def _():
                wait_c(s4)
            wait_a(s8)
            pltpu.async_copy(c0t_h.at[cb[s8].at[0, 0]], c0b[s4], semb[s4])
            pltpu.async_copy(c1t_h.at[cb[s8].at[1, 0]], c1b[s4], semb[s4])
            pltpu.async_copy(
                ce_h.at[pl.ds(base + c * _C, _C)], ceb[s4], semb[s4])

        def wait_b(s4):
            pltpu.make_async_copy(
                c0t_h.at[cb[0].at[0, 0]], c0b[s4], semb[s4]).wait()
            pltpu.make_async_copy(
                c1t_h.at[cb[0].at[1, 0]], c1b[s4], semb[s4]).wait()
            pltpu.make_async_copy(
                ce_h.at[pl.ds(0, _C)], ceb[s4], semb[s4]).wait()

        def compute(c, s8, s4):
            wait_b(s4)

            def grp(g2, c2):
                l0 = pl.multiple_of(g2 * 16, 16)
                i0vec = pb[s8][0, 0, pl.ds(l0, 16)]
                i1vec = pb[s8][1, 0, pl.ds(l0, 16)]
                for l in range(16):
                    t = l0 + l
                    id0 = i0vec[l]
                    id1 = i1vec[l]
                    for v in range(4):
                        plsc.addupdate(
                            c0b[s4].at[t, pl.ds(16 * v, 16)],
                            p0v[id0, pl.ds(16 * v, 16)])
                    for v in range(2):
                        plsc.addupdate(
                            c1b[s4].at[t, pl.ds(16 * v, 16)],
                            p1v[id1, pl.ds(16 * v, 16)])
                    for v in range(2):
                        plsc.addupdate(
                            ceb[s4].at[t, pl.ds(16 * v, 16)],
                            p1v[id1, pl.ds(32 + 16 * v, 16)])
                return c2

            lax.fori_loop(0, 8, grp, 0)
            off = base + c * _C
            pltpu.async_copy(
                c0b[s4], out_h.at[pl.ds(off, _C), pl.ds(0, 64)], semc[s4])
            pltpu.async_copy(
                c1b[s4], out_h.at[pl.ds(off, _C), pl.ds(64, 32)], semc[s4])
            pltpu.async_copy(
                ceb[s4], out_h.at[pl.ds(off, _C), pl.ds(96, 32)], semc[s4])

        for c0 in range(4):
            issue_a(c0, c0)
        issue_b(0, 0, 0)
        issue_b(1, 1, 1)

        def step(i, carry):
            for kk in range(8):
                c = 8 * i + kk

                @pl.when(c + 4 < _CHUNKS)
                def _():
                    issue_a(c + 4, (kk + 4) % 8)

                @pl.when(c + 2 < _CHUNKS)
                def _():
                    issue_b(c + 2, (kk + 2) % 8, (kk + 2) % 4)

                compute(c, kk, kk % 4)
            return carry

        lax.fori_loop(0, _CHUNKS // 8, step, 0)
        for s4 in range(4):
            wait_c(s4)

    return k(pids, cids, ce4, p0t, p1t, c0t, c1t)


def kernel(pos_ids_0, pos_ids_1, cat_ids_0, cat_ids_1, continuous_feature,
           pos_table_0, pos_table_1, cat_table_0, cat_table_1, W_cont, b_cont):
    pids = jnp.stack([pos_ids_0.reshape(_N), pos_ids_1.reshape(_N)]
                     ).astype(jnp.int32).reshape(2, _NBLK, 128)
    cids = jnp.stack([cat_ids_0.reshape(_N), cat_ids_1.reshape(_N)]
                     ).astype(jnp.int32).reshape(2, _NBLK, 128)
    ea = jnp.eye(8, 4, dtype=jnp.float32)
    eb = jnp.eye(8, 4, k=-4, dtype=jnp.float32)
    w = W_cont.astype(jnp.float32)
    wa = jnp.einsum("sv,kj->skvj", ea, w).reshape(128, 128)
    wb = jnp.einsum("sv,kj->skvj", eb, w).reshape(128, 128)
    b4 = jnp.tile(b_cont.astype(jnp.float32), 4).reshape(1, 128)
    ce4 = _cont_proj(continuous_feature.transpose(1, 2, 0), wa, wb, b4)
    out = _sc_assemble(pids, cids, ce4.reshape(_N, 32),
                       pos_table_0, pos_table_1, cat_table_0, cat_table_1)
    return out.reshape(_B, _S, 128)
